# baseline (speedup 1.0000x reference)
import functools

import numpy as np
import jax
import jax.numpy as jnp
from jax import lax
from jax.experimental import pallas as pl
from jax.experimental.pallas import tpu as pltpu
from jax.experimental.pallas import tpu_sc as plsc

VOCAB = 100000
EMB = 128
HID = 1024
TAGS = 1000
B = 4096
L = 200

NC = 2
NS = 16
NW = NC * NS
BPW = B // NW
LANES = 16
NCH = EMB // LANES

LCH = (48, 48, 48, 56)
LOFF = (0, 48, 96, 144)


NBUF = 4


def _pool_body(bpw, x_hbm, emb_hbm, out_hbm, *refs):
    idx = list(refs[0:NBUF])
    rows = list(refs[NBUF:2 * NBUF])
    acc = list(refs[2 * NBUF:3 * NBUF])
    isem = list(refs[3 * NBUF:4 * NBUF])
    gsem = list(refs[4 * NBUF:5 * NBUF])
    osem = list(refs[5 * NBUF:6 * NBUF])
    BPW = bpw
    wid = lax.axis_index("s") * NC + lax.axis_index("c")
    base = wid * BPW

    def fire_gather(s):
        for off, n in zip(LOFF, LCH):
            pltpu.async_copy(
                emb_hbm.at[idx[s].at[pl.ds(off, n)]],
                rows[s].at[pl.ds(off, n)], gsem[s])

    def wait_gather(s):
        pltpu.make_async_copy(emb_hbm.at[pl.ds(0, L)], rows[s], gsem[s]).wait()

    for s in range(NBUF - 1):
        pltpu.sync_copy(x_hbm.at[base + s], idx[s])
        fire_gather(s)
    pltpu.async_copy(x_hbm.at[base + NBUF - 1], idx[NBUF - 1], isem[NBUF - 1])

    @pl.loop(0, BPW, step=NBUF)
    def _outer(io):
        for s in range(NBUF):
            i = io + s
            nxt = (s + NBUF - 1) % NBUF
            wait_gather(s)

            @pl.when(i + NBUF - 1 < BPW)
            def _():
                pltpu.make_async_copy(
                    x_hbm.at[base + i + NBUF - 1], idx[nxt], isem[nxt]).wait()
                fire_gather(nxt)

            cfull = jnp.zeros((LANES,), jnp.float32)
            for k in range(L // LANES):
                v = idx[s][pl.ds(k * LANES, LANES)]
                cfull += jnp.where(v != 0, 1.0, 0.0)
            vt = idx[s][pl.ds(L - LANES, LANES)]
            lane = lax.iota(jnp.int32, LANES)
            tail = jnp.where((vt != 0) & (lane >= LANES - L % LANES), 1.0, 0.0)
            cnt = jnp.sum(cfull + tail)
            sv = 1.0 / (jnp.full((LANES,), cnt, jnp.float32) + 1e-05)

            @pl.when(i + NBUF < BPW)
            def _():
                pltpu.async_copy(x_hbm.at[base + i + NBUF], idx[s], isem[s])

            @pl.when(i >= NBUF)
            def _():
                pltpu.make_async_copy(
                    acc[s], out_hbm.at[base + i - NBUF], osem[s]).wait()

            zeros = tuple(jnp.zeros((LANES,), jnp.float32) for _ in range(NCH))

            @pl.loop(0, L, init_carry=zeros, unroll=2)
            def _acc(l, carry):
                return tuple(
                    c + rows[s][l, pl.ds(j * LANES, LANES)]
                    for j, c in enumerate(carry))

            for j in range(NCH):
                acc[s][pl.ds(j * LANES, LANES)] = _acc[j] * sv
            pltpu.async_copy(acc[s], out_hbm.at[base + i], osem[s])

    for s in range(NBUF):
        pltpu.make_async_copy(
            acc[s], out_hbm.at[base + BPW - NBUF + s], osem[s]).wait()


@functools.lru_cache(maxsize=None)
def _make_pool(nb):
    return pl.kernel(
        functools.partial(_pool_body, nb // NW),
        out_type=jax.ShapeDtypeStruct((nb, EMB), jnp.float32),
        mesh=plsc.VectorSubcoreMesh(core_axis_name="c", subcore_axis_name="s"),
        compiler_params=pltpu.CompilerParams(needs_layout_passes=False),
        scratch_types=(
            [pltpu.VMEM((L,), jnp.int32)] * NBUF
            + [pltpu.VMEM((L, EMB), jnp.float32)] * NBUF
            + [pltpu.VMEM((EMB,), jnp.float32)] * NBUF
            + [pltpu.SemaphoreType.DMA] * (3 * NBUF)
        ),
    )


BB = 256


def _mlp_body(p_ref, w1_ref, b1_ref, w2_ref, b2_ref, wf_ref, bf_ref, o_ref):
    h = p_ref[...].astype(jnp.bfloat16)
    h = lax.dot_general(h, w1_ref[...], (((1,), (1,)), ((), ())),
                        preferred_element_type=jnp.float32) + b1_ref[...]
    h = jnp.where(h > 0, h, 0.2 * h).astype(jnp.bfloat16)
    h = lax.dot_general(h, w2_ref[...], (((1,), (1,)), ((), ())),
                        preferred_element_type=jnp.float32) + b2_ref[...]
    h = jnp.where(h > 0, h, 0.2 * h).astype(jnp.bfloat16)
    o_ref[...] = lax.dot_general(h, wf_ref[...], (((1,), (1,)), ((), ())),
                                 preferred_element_type=jnp.float32) + bf_ref[...]


def _mlp(pooled, W1, b1, W2, b2, Wf, bf):
    nb = pooled.shape[0]
    grid = (nb // BB,)
    return pl.pallas_call(
        _mlp_body,
        grid=grid,
        in_specs=[
            pl.BlockSpec((BB, EMB), lambda i: (i, 0)),
            pl.BlockSpec((HID, EMB), lambda i: (0, 0)),
            pl.BlockSpec((1, HID), lambda i: (0, 0)),
            pl.BlockSpec((HID, HID), lambda i: (0, 0)),
            pl.BlockSpec((1, HID), lambda i: (0, 0)),
            pl.BlockSpec((TAGS, HID), lambda i: (0, 0)),
            pl.BlockSpec((1, TAGS), lambda i: (0, 0)),
        ],
        out_specs=pl.BlockSpec((BB, TAGS), lambda i: (i, 0)),
        out_shape=jax.ShapeDtypeStruct((nb, TAGS), jnp.float32),
    )(pooled, W1, b1, W2, b2, Wf, bf)


@jax.jit
def kernel(x, emb, W1, b1, W2, b2, Wf, bf):
    pooled = _make_pool(B)(x, emb)
    return _mlp(pooled, W1.astype(jnp.bfloat16), b1[None, :],
                W2.astype(jnp.bfloat16), b2[None, :],
                Wf.astype(jnp.bfloat16), bf[None, :])


# speedup vs baseline: 1.0186x; 1.0186x over previous
import functools

import numpy as np
import jax
import jax.numpy as jnp
from jax import lax
from jax.experimental import pallas as pl
from jax.experimental.pallas import tpu as pltpu
from jax.experimental.pallas import tpu_sc as plsc

VOCAB = 100000
EMB = 128
HID = 1024
TAGS = 1000
B = 4096
L = 200

NC = 2
NS = 16
NW = NC * NS
BPW = B // NW
LANES = 16
NCH = EMB // LANES

LCH = (48, 48, 48, 56)
LOFF = (0, 48, 96, 144)


NBUF = 4


def _pool_body(bpw, x_hbm, emb_hbm, out_hbm, *refs):
    idx = list(refs[0:NBUF])
    rows = list(refs[NBUF:2 * NBUF])
    acc = list(refs[2 * NBUF:3 * NBUF])
    isem = list(refs[3 * NBUF:4 * NBUF])
    gsem = list(refs[4 * NBUF:5 * NBUF])
    osem = list(refs[5 * NBUF:6 * NBUF])
    BPW = bpw
    wid = lax.axis_index("s") * NC + lax.axis_index("c")
    base = wid * BPW

    def fire_gather(s):
        for off, n in zip(LOFF, LCH):
            pltpu.async_copy(
                emb_hbm.at[idx[s].at[pl.ds(off, n)]],
                rows[s].at[pl.ds(off, n)], gsem[s])

    def wait_gather(s):
        pltpu.make_async_copy(emb_hbm.at[pl.ds(0, L)], rows[s], gsem[s]).wait()

    for s in range(NBUF - 1):
        pltpu.sync_copy(x_hbm.at[base + s], idx[s])
        fire_gather(s)
    pltpu.async_copy(x_hbm.at[base + NBUF - 1], idx[NBUF - 1], isem[NBUF - 1])

    @pl.loop(0, BPW, step=NBUF)
    def _outer(io):
        for s in range(NBUF):
            i = io + s
            nxt = (s + NBUF - 1) % NBUF
            wait_gather(s)

            @pl.when(i + NBUF - 1 < BPW)
            def _():
                pltpu.make_async_copy(
                    x_hbm.at[base + i + NBUF - 1], idx[nxt], isem[nxt]).wait()
                fire_gather(nxt)

            cfull = jnp.zeros((LANES,), jnp.float32)
            for k in range(L // LANES):
                v = idx[s][pl.ds(k * LANES, LANES)]
                cfull += jnp.where(v != 0, 1.0, 0.0)
            vt = idx[s][pl.ds(L - LANES, LANES)]
            lane = lax.iota(jnp.int32, LANES)
            tail = jnp.where((vt != 0) & (lane >= LANES - L % LANES), 1.0, 0.0)
            cnt = jnp.sum(cfull + tail)
            sv = 1.0 / (jnp.full((LANES,), cnt, jnp.float32) + 1e-05)

            @pl.when(i + NBUF < BPW)
            def _():
                pltpu.async_copy(x_hbm.at[base + i + NBUF], idx[s], isem[s])

            @pl.when(i >= NBUF)
            def _():
                pltpu.make_async_copy(
                    acc[s], out_hbm.at[base + i - NBUF], osem[s]).wait()

            zeros = tuple(jnp.zeros((LANES,), jnp.float32) for _ in range(NCH))

            @pl.loop(0, L, init_carry=zeros, unroll=2)
            def _acc(l, carry):
                return tuple(
                    c + rows[s][l, pl.ds(j * LANES, LANES)]
                    for j, c in enumerate(carry))

            for j in range(NCH):
                acc[s][pl.ds(j * LANES, LANES)] = _acc[j] * sv
            pltpu.async_copy(acc[s], out_hbm.at[base + i], osem[s])

    for s in range(NBUF):
        pltpu.make_async_copy(
            acc[s], out_hbm.at[base + BPW - NBUF + s], osem[s]).wait()


@functools.lru_cache(maxsize=None)
def _make_pool(nb):
    return pl.kernel(
        functools.partial(_pool_body, nb // NW),
        out_type=jax.ShapeDtypeStruct((nb, EMB), jnp.float32),
        mesh=plsc.VectorSubcoreMesh(core_axis_name="c", subcore_axis_name="s"),
        compiler_params=pltpu.CompilerParams(needs_layout_passes=False),
        scratch_types=(
            [pltpu.VMEM((L,), jnp.int32)] * NBUF
            + [pltpu.VMEM((L, EMB), jnp.float32)] * NBUF
            + [pltpu.VMEM((EMB,), jnp.float32)] * NBUF
            + [pltpu.SemaphoreType.DMA] * (3 * NBUF)
        ),
    )


BB = 1024


def _mlp_body(p_ref, w1_ref, b1_ref, w2_ref, b2_ref, wf_ref, bf_ref, o_ref):
    h = p_ref[...].astype(jnp.bfloat16)
    h = lax.dot_general(h, w1_ref[...], (((1,), (1,)), ((), ())),
                        preferred_element_type=jnp.float32) + b1_ref[...]
    h = jnp.where(h > 0, h, 0.2 * h).astype(jnp.bfloat16)
    h = lax.dot_general(h, w2_ref[...], (((1,), (1,)), ((), ())),
                        preferred_element_type=jnp.float32) + b2_ref[...]
    h = jnp.where(h > 0, h, 0.2 * h).astype(jnp.bfloat16)
    o_ref[...] = lax.dot_general(h, wf_ref[...], (((1,), (1,)), ((), ())),
                                 preferred_element_type=jnp.float32) + bf_ref[...]


def _mlp(pooled, W1, b1, W2, b2, Wf, bf):
    nb = pooled.shape[0]
    grid = (nb // BB,)
    return pl.pallas_call(
        _mlp_body,
        grid=grid,
        in_specs=[
            pl.BlockSpec((BB, EMB), lambda i: (i, 0)),
            pl.BlockSpec((HID, EMB), lambda i: (0, 0)),
            pl.BlockSpec((1, HID), lambda i: (0, 0)),
            pl.BlockSpec((HID, HID), lambda i: (0, 0)),
            pl.BlockSpec((1, HID), lambda i: (0, 0)),
            pl.BlockSpec((TAGS, HID), lambda i: (0, 0)),
            pl.BlockSpec((1, TAGS), lambda i: (0, 0)),
        ],
        out_specs=pl.BlockSpec((BB, TAGS), lambda i: (i, 0)),
        out_shape=jax.ShapeDtypeStruct((nb, TAGS), jnp.float32),
    )(pooled, W1, b1, W2, b2, Wf, bf)


@jax.jit
def kernel(x, emb, W1, b1, W2, b2, Wf, bf):
    pooled = _make_pool(B)(x, emb)
    return _mlp(pooled, W1.astype(jnp.bfloat16), b1[None, :],
                W2.astype(jnp.bfloat16), b2[None, :],
                Wf.astype(jnp.bfloat16), bf[None, :])
